# baseline (speedup 1.0000x reference)
import functools

import jax
import jax.numpy as jnp
from jax import lax
from jax.experimental import pallas as pl
from jax.experimental.pallas import tpu as pltpu
from jax.experimental.pallas import tpu_sc as plsc

T, N, C_IN, C_OUT, E = 12, 5000, 128, 16, 80000
H = C_OUT // 2
NP = 5008
HW = NP * H
CH = 8000
LANES = 16
UNROLL = 8


def _tc_body(x_ref, to_ref, w_ref, b_ref, wt_ref, ap_ref, h_ref, a2_ref):
    xt = x_ref[0]
    hT = lax.dot_general(w_ref[...], xt, (((1,), (1,)), ((), ())),
                         preferred_element_type=jnp.float32) + b_ref[...]
    hp = jnp.concatenate(
        [hT, jnp.zeros((C_OUT, NP - N), jnp.float32)], axis=1)
    h_ref[0] = hp
    te = lax.dot_general(to_ref[...], wt_ref[...], (((1,), (1,)), ((), ())),
                         preferred_element_type=jnp.float32)
    attn = jnp.dot(te, ap_ref[...],
                   preferred_element_type=jnp.float32)
    attn2 = jnp.concatenate([attn[:, :C_OUT], attn[:, C_OUT:]], axis=0)
    a2 = lax.dot_general(attn2, hT, (((1,), (0,)), ((), ())),
                         preferred_element_type=jnp.float32)
    m2 = jnp.max(a2, axis=1, keepdims=True)
    a2_ref[0] = jnp.concatenate(
        [a2, jnp.broadcast_to(m2, (2, NP - N))], axis=1)


def _tc_pre(x2, timeoh, W_mlp, b2, W_time, atten_pool):
    return pl.pallas_call(
        _tc_body,
        grid=(T,),
        in_specs=[
            pl.BlockSpec((1, N, C_IN), lambda t: (t, 0, 0)),
            pl.BlockSpec((1, 43), lambda t: (0, 0)),
            pl.BlockSpec((C_OUT, C_IN), lambda t: (0, 0)),
            pl.BlockSpec((C_OUT, 1), lambda t: (0, 0)),
            pl.BlockSpec((8, 43), lambda t: (0, 0)),
            pl.BlockSpec((8, 2 * C_OUT), lambda t: (0, 0)),
        ],
        out_specs=[
            pl.BlockSpec((1, C_OUT, NP), lambda t: (t, 0, 0)),
            pl.BlockSpec((1, 2, NP), lambda t: (t, 0, 0)),
        ],
        out_shape=[
            jax.ShapeDtypeStruct((T, C_OUT, NP), jnp.float32),
            jax.ShapeDtypeStruct((T, 2, NP), jnp.float32),
        ],
    )(x2, timeoh, W_mlp, b2, W_time, atten_pool)


NCH = E // CH
NBUF = 2


def _sc_body(h_hbm, a_hbm, src_hbm, dst_hbm, out_hbm,
             h_buf, out_buf, as_buf, ad_buf, rs_buf,
             src0, dst0, src1, dst1,
             sem_h, sem_as, sem_ad, sem_s0, sem_d0, sem_s1, sem_d1):
    cid = lax.axis_index("c")
    sid = lax.axis_index("s")
    t = sid
    half = cid
    bufs = ((src0, dst0, sem_s0, sem_d0), (src1, dst1, sem_s1, sem_d1))

    def start_chunk(i, b):
        off = pl.multiple_of(i * CH, 8)
        pltpu.async_copy(src_hbm.at[pl.ds(off, CH)], bufs[b][0], bufs[b][2])
        pltpu.async_copy(dst_hbm.at[pl.ds(off, CH)], bufs[b][1], bufs[b][3])

    def wait_chunk(b):
        pltpu.make_async_copy(src_hbm.at[pl.ds(0, CH)],
                              bufs[b][0], bufs[b][2]).wait()
        pltpu.make_async_copy(dst_hbm.at[pl.ds(0, CH)],
                              bufs[b][1], bufs[b][3]).wait()

    @pl.when(sid < T)
    def _run():
        cp_h = pltpu.async_copy(h_hbm.at[t, half], h_buf, sem_h)
        cp_as = pltpu.async_copy(a_hbm.at[t, 0], as_buf, sem_as)
        cp_ad = pltpu.async_copy(a_hbm.at[t, 1], ad_buf, sem_ad)
        start_chunk(0, 0)
        start_chunk(1, 1)

        zf = jnp.zeros((LANES,), jnp.float32)

        @plsc.parallel_loop(0, HW // LANES, unroll=8)
        def zero_out(i):
            out_buf[pl.ds(i * LANES, LANES)] = zf

        @plsc.parallel_loop(0, NP // LANES)
        def zero_rs(i):
            rs_buf[pl.ds(i * LANES, LANES)] = zf

        cp_h.wait()
        cp_as.wait()
        cp_ad.wait()

        mi = jnp.full((LANES,), N, jnp.int32)
        M = plsc.load_gather(as_buf, [mi]) + plsc.load_gather(ad_buf, [mi])

        def chunk_pair(k2, _):
            for b in range(NBUF):
                i = k2 * NBUF + b
                wait_chunk(b)
                src_buf, dst_buf = bufs[b][0], bufs[b][1]

                @plsc.parallel_loop(0, CH // LANES, unroll=UNROLL)
                def grp(g):
                    s_idx = src_buf[pl.ds(g * LANES, LANES)]
                    d_idx = dst_buf[pl.ds(g * LANES, LANES)]
                    av = (plsc.load_gather(as_buf, [s_idx])
                          + plsc.load_gather(ad_buf, [d_idx]))
                    lr = jnp.where(av >= 0.0, av, av * 0.01)
                    w = jnp.exp(lr - M)
                    plsc.addupdate_scatter(rs_buf, [s_idx], w)
                    for c in range(H):
                        col = plsc.load_gather(h_buf, [d_idx + c * NP])
                        plsc.addupdate_scatter(out_buf, [s_idx + c * NP],
                                               w * col)

                @pl.when(i + NBUF < NCH)
                def _prefetch():
                    start_chunk(i + NBUF, b)
            return None
        lax.fori_loop(0, NCH // NBUF, chunk_pair, None)

        @plsc.parallel_loop(0, NP // LANES)
        def div(i):
            rcp = 1.0 / (rs_buf[pl.ds(i * LANES, LANES)] + 9e-15)
            for c in range(H):
                off = pl.multiple_of(c * NP + i * LANES, 8)
                out_buf[pl.ds(off, LANES)] = out_buf[pl.ds(off, LANES)] * rcp

        pltpu.sync_copy(out_buf, out_hbm.at[t, half])


_sc_kernel = pl.kernel(
    _sc_body,
    out_type=jax.ShapeDtypeStruct((T, 2, HW), jnp.float32),
    mesh=plsc.VectorSubcoreMesh(core_axis_name="c", subcore_axis_name="s"),
    compiler_params=pltpu.CompilerParams(needs_layout_passes=False),
    scratch_types=[
        pltpu.VMEM((HW,), jnp.float32),
        pltpu.VMEM((HW,), jnp.float32),
        pltpu.VMEM((NP,), jnp.float32),
        pltpu.VMEM((NP,), jnp.float32),
        pltpu.VMEM((NP,), jnp.float32),
        pltpu.VMEM((CH,), jnp.int32),
        pltpu.VMEM((CH,), jnp.int32),
        pltpu.VMEM((CH,), jnp.int32),
        pltpu.VMEM((CH,), jnp.int32),
        pltpu.SemaphoreType.DMA,
        pltpu.SemaphoreType.DMA,
        pltpu.SemaphoreType.DMA,
        pltpu.SemaphoreType.DMA,
        pltpu.SemaphoreType.DMA,
        pltpu.SemaphoreType.DMA,
        pltpu.SemaphoreType.DMA,
    ],
)


@jax.jit
def kernel(x, timeoh, support, W_mlp, b_mlp, W_time, atten_pool):
    x2 = x[0]
    hp, a2 = _tc_pre(x2, timeoh, W_mlp, b_mlp.reshape(C_OUT, 1),
                     W_time, atten_pool)
    h_sc = hp.reshape(T, 2, HW)
    out_sc = _sc_kernel(h_sc, a2, support[0], support[1])
    out = out_sc.reshape(T, C_OUT, NP)[:, :, :N].transpose(0, 2, 1)
    return out[None]


# speedup vs baseline: 1.2030x; 1.2030x over previous
import functools

import jax
import jax.numpy as jnp
from jax import lax
from jax.experimental import pallas as pl
from jax.experimental.pallas import tpu as pltpu
from jax.experimental.pallas import tpu_sc as plsc

T, N, C_IN, C_OUT, E = 12, 5000, 128, 16, 80000
H = C_OUT // 2
NP = 5008
HW = NP * H
CH = 8000
LANES = 16
UNROLL = 4


def _tc_body(x_ref, to_ref, w_ref, b_ref, wt_ref, ap_ref, h_ref, a2_ref):
    xt = x_ref[0]
    hT = lax.dot_general(w_ref[...], xt, (((1,), (1,)), ((), ())),
                         preferred_element_type=jnp.float32) + b_ref[...]
    hp = jnp.concatenate(
        [hT, jnp.zeros((C_OUT, NP - N), jnp.float32)], axis=1)
    hu = lax.bitcast_convert_type(hp.astype(jnp.bfloat16),
                                  jnp.uint16).astype(jnp.uint32)
    hu = hu.reshape(C_OUT // 2, 2, NP)
    packed = hu[:, 0, :] | (hu[:, 1, :] << 16)
    h_ref[0] = lax.bitcast_convert_type(packed, jnp.float32)
    te = lax.dot_general(to_ref[...], wt_ref[...], (((1,), (1,)), ((), ())),
                         preferred_element_type=jnp.float32)
    attn = jnp.dot(te, ap_ref[...],
                   preferred_element_type=jnp.float32)
    attn2 = jnp.concatenate([attn[:, :C_OUT], attn[:, C_OUT:]], axis=0)
    a2 = lax.dot_general(attn2, hT, (((1,), (0,)), ((), ())),
                         preferred_element_type=jnp.float32)
    m2 = jnp.max(a2, axis=1, keepdims=True)
    a2_ref[0] = jnp.concatenate(
        [a2, jnp.broadcast_to(m2, (2, NP - N))], axis=1)


def _tc_pre(x2, timeoh, W_mlp, b2, W_time, atten_pool):
    return pl.pallas_call(
        _tc_body,
        grid=(T,),
        in_specs=[
            pl.BlockSpec((1, N, C_IN), lambda t: (t, 0, 0)),
            pl.BlockSpec((1, 43), lambda t: (0, 0)),
            pl.BlockSpec((C_OUT, C_IN), lambda t: (0, 0)),
            pl.BlockSpec((C_OUT, 1), lambda t: (0, 0)),
            pl.BlockSpec((8, 43), lambda t: (0, 0)),
            pl.BlockSpec((8, 2 * C_OUT), lambda t: (0, 0)),
        ],
        out_specs=[
            pl.BlockSpec((1, C_OUT // 2, NP), lambda t: (t, 0, 0)),
            pl.BlockSpec((1, 2, NP), lambda t: (t, 0, 0)),
        ],
        out_shape=[
            jax.ShapeDtypeStruct((T, C_OUT // 2, NP), jnp.float32),
            jax.ShapeDtypeStruct((T, 2, NP), jnp.float32),
        ],
    )(x2, timeoh, W_mlp, b2, W_time, atten_pool)


NCH = E // CH
NBUF = 2
HP = NP * H // 2


def _sc_body(h_hbm, a_hbm, src_hbm, dst_hbm, out_hbm,
             h_buf, out_buf, as_buf, ad_buf, rs_buf,
             src0, dst0, src1, dst1,
             sem_h, sem_as, sem_ad, sem_s0, sem_d0, sem_s1, sem_d1):
    cid = lax.axis_index("c")
    sid = lax.axis_index("s")
    t = sid
    half = cid
    bufs = ((src0, dst0, sem_s0, sem_d0), (src1, dst1, sem_s1, sem_d1))

    def start_chunk(i, b):
        off = pl.multiple_of(i * CH, 8)
        pltpu.async_copy(src_hbm.at[pl.ds(off, CH)], bufs[b][0], bufs[b][2])
        pltpu.async_copy(dst_hbm.at[pl.ds(off, CH)], bufs[b][1], bufs[b][3])

    def wait_chunk(b):
        pltpu.make_async_copy(src_hbm.at[pl.ds(0, CH)],
                              bufs[b][0], bufs[b][2]).wait()
        pltpu.make_async_copy(dst_hbm.at[pl.ds(0, CH)],
                              bufs[b][1], bufs[b][3]).wait()

    @pl.when(sid < T)
    def _run():
        cp_h = pltpu.async_copy(h_hbm.at[t, half], h_buf, sem_h)
        cp_as = pltpu.async_copy(a_hbm.at[t, 0], as_buf, sem_as)
        cp_ad = pltpu.async_copy(a_hbm.at[t, 1], ad_buf, sem_ad)
        start_chunk(0, 0)
        start_chunk(1, 1)

        zf = jnp.zeros((LANES,), jnp.float32)

        @plsc.parallel_loop(0, HW // LANES, unroll=8)
        def zero_out(i):
            out_buf[pl.ds(i * LANES, LANES)] = zf

        @plsc.parallel_loop(0, NP // LANES)
        def zero_rs(i):
            rs_buf[pl.ds(i * LANES, LANES)] = zf

        cp_h.wait()
        cp_as.wait()
        cp_ad.wait()

        mi = jnp.full((LANES,), N, jnp.int32)
        M = plsc.load_gather(as_buf, [mi]) + plsc.load_gather(ad_buf, [mi])

        def chunk_pair(k2, _):
            for b in range(NBUF):
                i = k2 * NBUF + b
                wait_chunk(b)
                src_buf, dst_buf = bufs[b][0], bufs[b][1]

                @plsc.parallel_loop(0, CH // LANES, unroll=UNROLL)
                def grp(g):
                    s_idx = src_buf[pl.ds(g * LANES, LANES)]
                    d_idx = dst_buf[pl.ds(g * LANES, LANES)]
                    av = (plsc.load_gather(as_buf, [s_idx])
                          + plsc.load_gather(ad_buf, [d_idx]))
                    lr = jnp.where(av >= 0.0, av, av * 0.01)
                    w = jnp.exp(lr - M)
                    plsc.addupdate_scatter(rs_buf, [s_idx], w)
                    for p in range(H // 2):
                        cu = plsc.bitcast(
                            plsc.load_gather(h_buf, [d_idx + p * NP]),
                            jnp.uint32)
                        lo = plsc.bitcast(cu << 16, jnp.float32)
                        hi = plsc.bitcast(cu & jnp.uint32(0xFFFF0000),
                                          jnp.float32)
                        plsc.addupdate_scatter(
                            out_buf, [s_idx + (2 * p) * NP], w * lo)
                        plsc.addupdate_scatter(
                            out_buf, [s_idx + (2 * p + 1) * NP], w * hi)

                @pl.when(i + NBUF < NCH)
                def _prefetch():
                    start_chunk(i + NBUF, b)
            return None
        lax.fori_loop(0, NCH // NBUF, chunk_pair, None)

        @plsc.parallel_loop(0, NP // LANES)
        def div(i):
            rcp = 1.0 / (rs_buf[pl.ds(i * LANES, LANES)] + 9e-15)
            for c in range(H):
                off = pl.multiple_of(c * NP + i * LANES, 8)
                out_buf[pl.ds(off, LANES)] = out_buf[pl.ds(off, LANES)] * rcp

        pltpu.sync_copy(out_buf, out_hbm.at[t, half])


_sc_kernel = pl.kernel(
    _sc_body,
    out_type=jax.ShapeDtypeStruct((T, 2, HW), jnp.float32),
    mesh=plsc.VectorSubcoreMesh(core_axis_name="c", subcore_axis_name="s"),
    compiler_params=pltpu.CompilerParams(needs_layout_passes=False),
    scratch_types=[
        pltpu.VMEM((HP,), jnp.float32),
        pltpu.VMEM((HW,), jnp.float32),
        pltpu.VMEM((NP,), jnp.float32),
        pltpu.VMEM((NP,), jnp.float32),
        pltpu.VMEM((NP,), jnp.float32),
        pltpu.VMEM((CH,), jnp.int32),
        pltpu.VMEM((CH,), jnp.int32),
        pltpu.VMEM((CH,), jnp.int32),
        pltpu.VMEM((CH,), jnp.int32),
        pltpu.SemaphoreType.DMA,
        pltpu.SemaphoreType.DMA,
        pltpu.SemaphoreType.DMA,
        pltpu.SemaphoreType.DMA,
        pltpu.SemaphoreType.DMA,
        pltpu.SemaphoreType.DMA,
        pltpu.SemaphoreType.DMA,
    ],
)


@jax.jit
def kernel(x, timeoh, support, W_mlp, b_mlp, W_time, atten_pool):
    x2 = x[0]
    hp, a2 = _tc_pre(x2, timeoh, W_mlp, b_mlp.reshape(C_OUT, 1),
                     W_time, atten_pool)
    h_sc = hp.reshape(T, 2, HP)
    out_sc = _sc_kernel(h_sc, a2, support[0], support[1])
    out = out_sc.reshape(T, C_OUT, NP)[:, :, :N].transpose(0, 2, 1)
    return out[None]


# speedup vs baseline: 1.8922x; 1.5730x over previous
import functools

import jax
import jax.numpy as jnp
from jax import lax
from jax.experimental import pallas as pl
from jax.experimental.pallas import tpu as pltpu
from jax.experimental.pallas import tpu_sc as plsc

T, N, C_IN, C_OUT, E = 12, 5000, 128, 16, 80000
H = C_OUT // 2
NP = 5008
HW = NP * H
CH = 8000
LANES = 16
UNROLL = 4


def _tc_body(x_ref, to_ref, w_ref, b_ref, wt_ref, ap_ref, h_ref, a2_ref):
    xt = x_ref[0]
    hT = lax.dot_general(w_ref[...], xt, (((1,), (1,)), ((), ())),
                         preferred_element_type=jnp.float32) + b_ref[...]
    hp = jnp.concatenate(
        [hT, jnp.zeros((C_OUT, NP - N), jnp.float32)], axis=1)
    h_ref[0] = hp
    te = lax.dot_general(to_ref[...], wt_ref[...], (((1,), (1,)), ((), ())),
                         preferred_element_type=jnp.float32)
    attn = jnp.dot(te, ap_ref[...],
                   preferred_element_type=jnp.float32)
    attn2 = jnp.concatenate([attn[:, :C_OUT], attn[:, C_OUT:]], axis=0)
    a2 = lax.dot_general(attn2, hT, (((1,), (0,)), ((), ())),
                         preferred_element_type=jnp.float32)
    m2 = jnp.max(a2, axis=1, keepdims=True)
    a2_ref[0] = jnp.concatenate(
        [a2, jnp.broadcast_to(m2, (2, NP - N))], axis=1)


def _tc_pre(x2, timeoh, W_mlp, b2, W_time, atten_pool):
    return pl.pallas_call(
        _tc_body,
        grid=(T,),
        in_specs=[
            pl.BlockSpec((1, N, C_IN), lambda t: (t, 0, 0)),
            pl.BlockSpec((1, 43), lambda t: (0, 0)),
            pl.BlockSpec((C_OUT, C_IN), lambda t: (0, 0)),
            pl.BlockSpec((C_OUT, 1), lambda t: (0, 0)),
            pl.BlockSpec((8, 43), lambda t: (0, 0)),
            pl.BlockSpec((8, 2 * C_OUT), lambda t: (0, 0)),
        ],
        out_specs=[
            pl.BlockSpec((1, C_OUT, NP), lambda t: (t, 0, 0)),
            pl.BlockSpec((1, 2, NP), lambda t: (t, 0, 0)),
        ],
        out_shape=[
            jax.ShapeDtypeStruct((T, C_OUT, NP), jnp.float32),
            jax.ShapeDtypeStruct((T, 2, NP), jnp.float32),
        ],
    )(x2, timeoh, W_mlp, b2, W_time, atten_pool)


NCH = E // CH
NBUF = 2


def _sc_body(h_hbm, a_hbm, src_hbm, dst_hbm, out_hbm,
             h_buf, out_buf, as_buf, ad_buf, rs_buf,
             src0, dst0, src1, dst1,
             sem_h, sem_as, sem_ad, sem_s0, sem_d0, sem_s1, sem_d1):
    cid = lax.axis_index("c")
    sid = lax.axis_index("s")
    t = sid
    half = cid
    bufs = ((src0, dst0, sem_s0, sem_d0), (src1, dst1, sem_s1, sem_d1))

    def start_chunk(i, b):
        off = pl.multiple_of(i * CH, 8)
        pltpu.async_copy(src_hbm.at[pl.ds(off, CH)], bufs[b][0], bufs[b][2])
        pltpu.async_copy(dst_hbm.at[pl.ds(off, CH)], bufs[b][1], bufs[b][3])

    def wait_chunk(b):
        pltpu.make_async_copy(src_hbm.at[pl.ds(0, CH)],
                              bufs[b][0], bufs[b][2]).wait()
        pltpu.make_async_copy(dst_hbm.at[pl.ds(0, CH)],
                              bufs[b][1], bufs[b][3]).wait()

    @pl.when(sid < T)
    def _run():
        cp_h = pltpu.async_copy(h_hbm.at[t, half], h_buf, sem_h)
        cp_as = pltpu.async_copy(a_hbm.at[t, 0], as_buf, sem_as)
        cp_ad = pltpu.async_copy(a_hbm.at[t, 1], ad_buf, sem_ad)
        start_chunk(0, 0)
        start_chunk(1, 1)

        zf = jnp.zeros((LANES,), jnp.float32)

        @plsc.parallel_loop(0, HW // LANES, unroll=8)
        def zero_out(i):
            out_buf[pl.ds(i * LANES, LANES)] = zf

        @plsc.parallel_loop(0, NP // LANES)
        def zero_rs(i):
            rs_buf[pl.ds(i * LANES, LANES)] = zf

        cp_h.wait()
        cp_as.wait()
        cp_ad.wait()

        mi = jnp.full((LANES,), N, jnp.int32)
        M = plsc.load_gather(as_buf, [mi]) + plsc.load_gather(ad_buf, [mi])

        def chunk_pair(k2, _):
            for b in range(NBUF):
                i = k2 * NBUF + b
                wait_chunk(b)
                src_buf, dst_buf = bufs[b][0], bufs[b][1]

                @plsc.parallel_loop(0, CH // LANES, unroll=UNROLL)
                def grp(g):
                    s_idx = src_buf[pl.ds(g * LANES, LANES)]
                    d_idx = dst_buf[pl.ds(g * LANES, LANES)]
                    av = (plsc.load_gather(as_buf, [s_idx])
                          + plsc.load_gather(ad_buf, [d_idx]))
                    lr = jnp.where(av >= 0.0, av, av * 0.01)
                    w = jnp.exp(lr - M)
                    plsc.addupdate_scatter(rs_buf, [s_idx], w)
                    for c in range(H):
                        col = plsc.load_gather(h_buf, [d_idx + c * NP])
                        plsc.addupdate_scatter(out_buf, [s_idx + c * NP],
                                               w * col)

                @pl.when(i + NBUF < NCH)
                def _prefetch():
                    start_chunk(i + NBUF, b)
            return None
        lax.fori_loop(0, NCH // NBUF, chunk_pair, None)

        @plsc.parallel_loop(0, NP // LANES)
        def div(i):
            rcp = 1.0 / (rs_buf[pl.ds(i * LANES, LANES)] + 9e-15)
            for c in range(H):
                off = pl.multiple_of(c * NP + i * LANES, 8)
                out_buf[pl.ds(off, LANES)] = out_buf[pl.ds(off, LANES)] * rcp

        pltpu.sync_copy(out_buf, out_hbm.at[t, half])


_sc_kernel = pl.kernel(
    _sc_body,
    out_type=jax.ShapeDtypeStruct((T, 2, HW), jnp.float32),
    mesh=plsc.VectorSubcoreMesh(core_axis_name="c", subcore_axis_name="s"),
    compiler_params=pltpu.CompilerParams(needs_layout_passes=False),
    scratch_types=[
        pltpu.VMEM((HW,), jnp.float32),
        pltpu.VMEM((HW,), jnp.float32),
        pltpu.VMEM((NP,), jnp.float32),
        pltpu.VMEM((NP,), jnp.float32),
        pltpu.VMEM((NP,), jnp.float32),
        pltpu.VMEM((CH,), jnp.int32),
        pltpu.VMEM((CH,), jnp.int32),
        pltpu.VMEM((CH,), jnp.int32),
        pltpu.VMEM((CH,), jnp.int32),
        pltpu.SemaphoreType.DMA,
        pltpu.SemaphoreType.DMA,
        pltpu.SemaphoreType.DMA,
        pltpu.SemaphoreType.DMA,
        pltpu.SemaphoreType.DMA,
        pltpu.SemaphoreType.DMA,
        pltpu.SemaphoreType.DMA,
    ],
)


@jax.jit
def kernel(x, timeoh, support, W_mlp, b_mlp, W_time, atten_pool):
    x2 = x[0]
    hp, a2 = _tc_pre(x2, timeoh, W_mlp, b_mlp.reshape(C_OUT, 1),
                     W_time, atten_pool)
    h_sc = hp.reshape(T, 2, HW)
    out_sc = _sc_kernel(h_sc, a2, support[0], support[1])
    out = out_sc.reshape(T, C_OUT, NP)[:, :, :N].transpose(0, 2, 1)
    return out[None]
